# Initial kernel scaffold; baseline (speedup 1.0000x reference)
import jax
import jax.numpy as jnp
from jax.experimental import pallas as pl


def kernel(pos, v_neg, u_weight, v_weight):
    raise NotImplementedError("write your pallas kernel here")


# speedup vs baseline: 4.7433x; 4.7433x over previous
import functools

import jax
import jax.numpy as jnp
from jax import lax
from jax.experimental import pallas as pl
from jax.experimental.pallas import tpu as pltpu
from jax.experimental.pallas import tpu_sc as plsc

VOCAB = 100000
DIM = 128
BATCH = 4096
N_NEG = 20
LANES = 16
NC = 2
NS = 16
NW = NC * NS
BPW = BATCH // NW
CHUNKS = DIM // LANES
GROUPS = BPW // LANES


def _sc_scores(u_idx, v_idx, neg3, u_weight, v_weight):
    mesh = plsc.VectorSubcoreMesh(core_axis_name="c", subcore_axis_name="s")

    @functools.partial(
        pl.kernel,
        out_type=(
            jax.ShapeDtypeStruct((BATCH,), jnp.float32),
            jax.ShapeDtypeStruct((BATCH,), jnp.float32),
        ),
        mesh=mesh,
        compiler_params=pltpu.CompilerParams(needs_layout_passes=False),
        scratch_types=[
            pltpu.VMEM((BPW,), jnp.int32),
            pltpu.VMEM((BPW,), jnp.int32),
            pltpu.VMEM((N_NEG, BPW), jnp.int32),
            pltpu.VMEM((BPW, DIM), jnp.float32),
            pltpu.VMEM((BPW, DIM), jnp.float32),
            pltpu.VMEM((BPW, DIM), jnp.float32),
            pltpu.VMEM((BPW, DIM), jnp.float32),
            pltpu.VMEM((BPW,), jnp.float32),
            pltpu.VMEM((BPW,), jnp.float32),
            pltpu.SemaphoreType.DMA,
            pltpu.SemaphoreType.DMA,
        ],
    )
    def scores_kernel(u_idx_hbm, v_idx_hbm, neg3_hbm, u_w, v_w,
                      score_hbm, nscore_hbm,
                      uidx_v, vidx_v, nidx_v, u_rows, v_rows, negsum, negbuf,
                      score_v, nscore_v, sem0, sem1):
        wid = lax.axis_index("s") * NC + lax.axis_index("c")
        base = pl.multiple_of(wid * BPW, BPW)

        pltpu.sync_copy(u_idx_hbm.at[pl.ds(base, BPW)], uidx_v)
        pltpu.sync_copy(v_idx_hbm.at[pl.ds(base, BPW)], vidx_v)
        pltpu.sync_copy(neg3_hbm.at[wid], nidx_v)

        cu = pltpu.async_copy(u_w.at[uidx_v], u_rows, sem0)
        cv = pltpu.async_copy(v_w.at[vidx_v], v_rows, sem1)
        c0 = pltpu.async_copy(v_w.at[nidx_v.at[0]], negsum, sem0)
        cu.wait()
        cv.wait()
        c0.wait()

        for n in range(1, N_NEG):
            pltpu.async_copy(v_w.at[nidx_v.at[n]], negbuf, sem0).wait()

            def acc_body(i, carry):
                for c in range(CHUNKS):
                    sl = pl.ds(c * LANES, LANES)
                    negsum[i, sl] = negsum[i, sl] + negbuf[i, sl]
                return carry

            lax.fori_loop(0, BPW, acc_body, 0)

        lane_iota = jnp.arange(LANES, dtype=jnp.int32)

        def group_body(g, carry):
            sp = jnp.zeros((LANES,), jnp.float32)
            sn = jnp.zeros((LANES,), jnp.float32)
            for i in range(LANES):
                item = g * LANES + i
                accp = jnp.zeros((LANES,), jnp.float32)
                accn = jnp.zeros((LANES,), jnp.float32)
                for c in range(CHUNKS):
                    sl = pl.ds(c * LANES, LANES)
                    uu = u_rows[item, sl]
                    accp = accp + uu * v_rows[item, sl]
                    accn = accn + uu * negsum[item, sl]
                m = lane_iota == i
                sp = jnp.where(m, jnp.sum(accp), sp)
                sn = jnp.where(m, jnp.sum(accn), sn)
            score_v[pl.ds(g * LANES, LANES)] = sp
            nscore_v[pl.ds(g * LANES, LANES)] = sn
            return carry

        lax.fori_loop(0, GROUPS, group_body, 0)

        pltpu.sync_copy(score_v, score_hbm.at[pl.ds(base, BPW)])
        pltpu.sync_copy(nscore_v, nscore_hbm.at[pl.ds(base, BPW)])

    return scores_kernel(u_idx, v_idx, neg3, u_weight, v_weight)


def _loss_tc_kernel(s_ref, n_ref, o_ref):
    s = s_ref[...]
    ns = n_ref[...]

    def logsig(x):
        return jnp.minimum(x, 0.0) - jnp.log1p(jnp.exp(-jnp.abs(x)))

    total = jnp.sum(logsig(s) + logsig(-ns))
    o_ref[...] = jnp.full((1, 1), -total / BATCH, jnp.float32)


def kernel(pos, v_neg, u_weight, v_weight):
    u_idx = pos[:, 0].astype(jnp.int32)
    v_idx = pos[:, 1].astype(jnp.int32)
    neg3 = (
        v_neg.astype(jnp.int32)
        .reshape(NW, BPW, N_NEG)
        .transpose(0, 2, 1)
    )
    score, nscore = _sc_scores(u_idx, v_idx, neg3, u_weight, v_weight)

    out = pl.pallas_call(
        _loss_tc_kernel,
        out_shape=jax.ShapeDtypeStruct((1, 1), jnp.float32),
    )(score.reshape(NW, BPW), nscore.reshape(NW, BPW))
    return out[0, 0]


# speedup vs baseline: 5.9829x; 1.2613x over previous
import functools

import jax
import jax.numpy as jnp
from jax import lax
from jax.experimental import pallas as pl
from jax.experimental.pallas import tpu as pltpu
from jax.experimental.pallas import tpu_sc as plsc

VOCAB = 100000
DIM = 128
BATCH = 4096
N_NEG = 20
LANES = 16
NC = 2
NS = 16
NW = NC * NS
BPW = BATCH // NW
CHUNKS = DIM // LANES
GROUPS = BPW // LANES


def _sc_scores(u_idx, v_idx, neg3, u_weight, v_weight):
    mesh = plsc.VectorSubcoreMesh(core_axis_name="c", subcore_axis_name="s")

    @functools.partial(
        pl.kernel,
        out_type=(
            jax.ShapeDtypeStruct((BATCH,), jnp.float32),
            jax.ShapeDtypeStruct((BATCH,), jnp.float32),
        ),
        mesh=mesh,
        compiler_params=pltpu.CompilerParams(needs_layout_passes=False),
        scratch_types=[
            pltpu.VMEM((BPW,), jnp.int32),
            pltpu.VMEM((BPW,), jnp.int32),
            pltpu.VMEM((N_NEG, BPW), jnp.int32),
            pltpu.VMEM((BPW, DIM), jnp.float32),
            pltpu.VMEM((BPW, DIM), jnp.float32),
            pltpu.VMEM((BPW, DIM), jnp.float32),
            pltpu.VMEM((BPW, DIM), jnp.float32),
            pltpu.VMEM((BPW, DIM), jnp.float32),
            pltpu.VMEM((BPW,), jnp.float32),
            pltpu.VMEM((BPW,), jnp.float32),
            pltpu.SemaphoreType.DMA,
            pltpu.SemaphoreType.DMA,
            pltpu.SemaphoreType.DMA,
            pltpu.SemaphoreType.DMA,
        ],
    )
    def scores_kernel(u_idx_hbm, v_idx_hbm, neg3_hbm, u_w, v_w,
                      score_hbm, nscore_hbm,
                      uidx_v, vidx_v, nidx_v, u_rows, v_rows, negsum,
                      nbufa, nbufb, score_v, nscore_v,
                      semu, semv, sema, semb):
        wid = lax.axis_index("s") * NC + lax.axis_index("c")
        base = pl.multiple_of(wid * BPW, BPW)

        pltpu.sync_copy(u_idx_hbm.at[pl.ds(base, BPW)], uidx_v)
        pltpu.sync_copy(v_idx_hbm.at[pl.ds(base, BPW)], vidx_v)
        pltpu.sync_copy(neg3_hbm.at[wid], nidx_v)

        cu = pltpu.async_copy(u_w.at[uidx_v], u_rows, semu)
        cv = pltpu.async_copy(v_w.at[vidx_v], v_rows, semv)
        c0 = pltpu.async_copy(v_w.at[nidx_v.at[0]], negsum, sema)
        ca = pltpu.async_copy(v_w.at[nidx_v.at[1]], nbufa, semb)

        lane_iota = jnp.arange(LANES, dtype=jnp.int32)

        cu.wait()
        cv.wait()

        def pos_body(g, carry):
            sp = jnp.zeros((LANES,), jnp.float32)
            for i in range(LANES):
                item = g * LANES + i
                accp = jnp.zeros((LANES,), jnp.float32)
                for c in range(CHUNKS):
                    sl = pl.ds(c * LANES, LANES)
                    accp = accp + u_rows[item, sl] * v_rows[item, sl]
                sp = jnp.where(lane_iota == i, jnp.sum(accp), sp)
            score_v[pl.ds(g * LANES, LANES)] = sp
            return carry

        lax.fori_loop(0, GROUPS, pos_body, 0)
        pltpu.sync_copy(score_v, score_hbm.at[pl.ds(base, BPW)])

        c0.wait()
        bufs = (nbufa, nbufb)
        sems = (sema, semb)
        pend = ca
        for n in range(1, N_NEG):
            pend.wait()
            if n + 1 < N_NEG:
                pend = pltpu.async_copy(
                    v_w.at[nidx_v.at[n + 1]], bufs[n % 2], sems[n % 2]
                )
            cur = bufs[(n - 1) % 2]

            def acc_body(i, carry):
                for c in range(CHUNKS):
                    sl = pl.ds(c * LANES, LANES)
                    plsc.addupdate(negsum.at[i, sl], cur[i, sl])
                return carry

            lax.fori_loop(0, BPW, acc_body, 0)

        def neg_body(g, carry):
            sn = jnp.zeros((LANES,), jnp.float32)
            for i in range(LANES):
                item = g * LANES + i
                accn = jnp.zeros((LANES,), jnp.float32)
                for c in range(CHUNKS):
                    sl = pl.ds(c * LANES, LANES)
                    accn = accn + u_rows[item, sl] * negsum[item, sl]
                sn = jnp.where(lane_iota == i, jnp.sum(accn), sn)
            nscore_v[pl.ds(g * LANES, LANES)] = sn
            return carry

        lax.fori_loop(0, GROUPS, neg_body, 0)
        pltpu.sync_copy(nscore_v, nscore_hbm.at[pl.ds(base, BPW)])

    return scores_kernel(u_idx, v_idx, neg3, u_weight, v_weight)


def _loss_tc_kernel(s_ref, n_ref, o_ref):
    s = s_ref[...]
    ns = n_ref[...]

    def logsig(x):
        return jnp.minimum(x, 0.0) - jnp.log1p(jnp.exp(-jnp.abs(x)))

    total = jnp.sum(logsig(s) + logsig(-ns))
    o_ref[...] = jnp.full((1, 1), -total / BATCH, jnp.float32)


def kernel(pos, v_neg, u_weight, v_weight):
    u_idx = pos[:, 0].astype(jnp.int32)
    v_idx = pos[:, 1].astype(jnp.int32)
    neg3 = (
        v_neg.astype(jnp.int32)
        .reshape(NW, BPW, N_NEG)
        .transpose(0, 2, 1)
    )
    score, nscore = _sc_scores(u_idx, v_idx, neg3, u_weight, v_weight)

    out = pl.pallas_call(
        _loss_tc_kernel,
        out_shape=jax.ShapeDtypeStruct((1, 1), jnp.float32),
    )(score.reshape(NW, BPW), nscore.reshape(NW, BPW))
    return out[0, 0]


# speedup vs baseline: 6.5867x; 1.1009x over previous
import functools

import jax
import jax.numpy as jnp
from jax import lax
from jax.experimental import pallas as pl
from jax.experimental.pallas import tpu as pltpu
from jax.experimental.pallas import tpu_sc as plsc

VOCAB = 100000
DIM = 128
BATCH = 4096
N_NEG = 20
LANES = 16
NC = 2
NS = 16
NW = NC * NS
BPW = BATCH // NW
CHUNKS = DIM // LANES
GROUPS = BPW // LANES


def _sc_scores(u_idx, v_idx, neg3, u_weight, v_weight):
    mesh = plsc.VectorSubcoreMesh(core_axis_name="c", subcore_axis_name="s")

    @functools.partial(
        pl.kernel,
        out_type=(
            jax.ShapeDtypeStruct((BATCH,), jnp.float32),
            jax.ShapeDtypeStruct((BATCH,), jnp.float32),
        ),
        mesh=mesh,
        compiler_params=pltpu.CompilerParams(needs_layout_passes=False),
        scratch_types=[
            pltpu.VMEM((BPW,), jnp.int32),
            pltpu.VMEM((BPW,), jnp.int32),
            pltpu.VMEM((N_NEG, BPW), jnp.int32),
            pltpu.VMEM((1, BPW), jnp.int32),
            pltpu.VMEM((BPW, DIM), jnp.float32),
            pltpu.VMEM((BPW, DIM), jnp.float32),
            pltpu.VMEM((BPW, DIM), jnp.float32),
            pltpu.VMEM_SHARED((NS * BPW, DIM), jnp.float32),
            pltpu.VMEM((BPW, DIM), jnp.float32),
            pltpu.VMEM((BPW, DIM), jnp.float32),
            pltpu.VMEM((BPW, DIM), jnp.float32),
            pltpu.VMEM((BPW,), jnp.float32),
            pltpu.VMEM((BPW,), jnp.float32),
            pltpu.SemaphoreType.DMA,
            pltpu.SemaphoreType.DMA,
            pltpu.SemaphoreType.DMA,
            pltpu.SemaphoreType.DMA,
            pltpu.SemaphoreType.DMA,
            pltpu.SemaphoreType.DMA,
            pltpu.SemaphoreType.DMA,
            pltpu.SemaphoreType.DMA,
            pltpu.SemaphoreType.DMA,
        ],
    )
    def scores_kernel(u_idx_hbm, v_idx_hbm, neg3_hbm, u_w, v_w,
                      score_hbm, nscore_hbm,
                      uidx_v, vidx_v, nidx_v, ident_v, u_rows, v_rows, negsum,
                      negsh, rb0, rb1, rb2, score_v, nscore_v,
                      semu, semv, sem0,
                      gs0, gs1, gs2, ss0, ss1, ss2):
        sub = lax.axis_index("s")
        wid = sub * NC + lax.axis_index("c")
        base = pl.multiple_of(wid * BPW, BPW)
        shbase = pl.multiple_of(sub * BPW, BPW)

        bufs = (rb0, rb1, rb2)
        gsems = (gs0, gs1, gs2)
        ssems = (ss0, ss1, ss2)
        NRING = 3

        pltpu.sync_copy(u_idx_hbm.at[pl.ds(base, BPW)], uidx_v)
        pltpu.sync_copy(v_idx_hbm.at[pl.ds(base, BPW)], vidx_v)
        pltpu.sync_copy(neg3_hbm.at[wid], nidx_v)

        lane_iota = jnp.arange(LANES, dtype=jnp.int32)
        for g in range(GROUPS):
            ident_v[0, pl.ds(g * LANES, LANES)] = shbase + g * LANES + lane_iota

        cu = pltpu.async_copy(u_w.at[uidx_v], u_rows, semu)
        cv = pltpu.async_copy(v_w.at[vidx_v], v_rows, semv)
        c0 = pltpu.async_copy(v_w.at[nidx_v.at[0]], negsum, sem0)
        gpend = {}
        for n in range(1, 1 + NRING):
            b = (n - 1) % NRING
            gpend[b] = pltpu.async_copy(v_w.at[nidx_v.at[n]], bufs[b], gsems[b])

        cu.wait()
        cv.wait()

        def pos_body(g, carry):
            sp = jnp.zeros((LANES,), jnp.float32)
            for i in range(LANES):
                item = g * LANES + i
                accp = jnp.zeros((LANES,), jnp.float32)
                for c in range(CHUNKS):
                    sl = pl.ds(c * LANES, LANES)
                    accp = accp + u_rows[item, sl] * v_rows[item, sl]
                sp = jnp.where(lane_iota == i, jnp.sum(accp), sp)
            score_v[pl.ds(g * LANES, LANES)] = sp
            return carry

        lax.fori_loop(0, GROUPS, pos_body, 0)
        pltpu.sync_copy(score_v, score_hbm.at[pl.ds(base, BPW)])

        c0.wait()
        pltpu.sync_copy(negsum, negsh.at[pl.ds(shbase, BPW)])
        spend = {}
        for n in range(1, N_NEG):
            b = (n - 1) % NRING
            gpend[b].wait()
            spend[b] = pltpu.async_copy(
                bufs[b], negsh.at[ident_v.at[0]], ssems[b], add=True
            )
            nxt = n + NRING
            if nxt < N_NEG:
                spend[b].wait()
                del spend[b]
                gpend[b] = pltpu.async_copy(
                    v_w.at[nidx_v.at[nxt]], bufs[b], gsems[b]
                )
        for b in sorted(spend):
            spend[b].wait()
        pltpu.sync_copy(negsh.at[pl.ds(shbase, BPW)], negsum)

        def neg_body(g, carry):
            sn = jnp.zeros((LANES,), jnp.float32)
            for i in range(LANES):
                item = g * LANES + i
                accn = jnp.zeros((LANES,), jnp.float32)
                for c in range(CHUNKS):
                    sl = pl.ds(c * LANES, LANES)
                    accn = accn + u_rows[item, sl] * negsum[item, sl]
                sn = jnp.where(lane_iota == i, jnp.sum(accn), sn)
            nscore_v[pl.ds(g * LANES, LANES)] = sn
            return carry

        lax.fori_loop(0, GROUPS, neg_body, 0)
        pltpu.sync_copy(nscore_v, nscore_hbm.at[pl.ds(base, BPW)])

    return scores_kernel(u_idx, v_idx, neg3, u_weight, v_weight)


def _loss_tc_kernel(s_ref, n_ref, o_ref):
    s = s_ref[...]
    ns = n_ref[...]

    def logsig(x):
        return jnp.minimum(x, 0.0) - jnp.log1p(jnp.exp(-jnp.abs(x)))

    total = jnp.sum(logsig(s) + logsig(-ns))
    o_ref[...] = jnp.full((1, 1), -total / BATCH, jnp.float32)


def kernel(pos, v_neg, u_weight, v_weight):
    u_idx = pos[:, 0].astype(jnp.int32)
    v_idx = pos[:, 1].astype(jnp.int32)
    neg3 = (
        v_neg.astype(jnp.int32)
        .reshape(NW, BPW, N_NEG)
        .transpose(0, 2, 1)
    )
    score, nscore = _sc_scores(u_idx, v_idx, neg3, u_weight, v_weight)

    out = pl.pallas_call(
        _loss_tc_kernel,
        out_shape=jax.ShapeDtypeStruct((1, 1), jnp.float32),
    )(score.reshape(NW, BPW), nscore.reshape(NW, BPW))
    return out[0, 0]
